# Initial kernel scaffold; baseline (speedup 1.0000x reference)
import jax
import jax.numpy as jnp
from jax.experimental import pallas as pl


def kernel(z_e, codebook):
    raise NotImplementedError("write your pallas kernel here")


# speedup vs baseline: 1.0141x; 1.0141x over previous
import jax
import jax.numpy as jnp
from jax.experimental import pallas as pl
from jax.experimental.pallas import tpu as pltpu

_N_CODES = 8192
_CODE_DIM = 256
_BETA = 0.25
_T_BLK = 1024
_K_BLK = 1024


def _vq_body(z_ref, cb_ref, zq_ref, idx_ref, dsum_ref):
    z = z_ref[...]
    zsq = jnp.sum(z * z, axis=1, keepdims=True)

    n_tiles = _N_CODES // _K_BLK

    def dist_step(kt, carry):
        best_d, best_i = carry
        koff = kt * _K_BLK
        c = cb_ref[pl.ds(koff, _K_BLK), :]
        csq = jnp.sum(c * c, axis=1)
        mm = jax.lax.dot_general(
            z, c, (((1,), (1,)), ((), ())),
            preferred_element_type=jnp.float32)
        d = (zsq - 2.0 * mm) + csq[None, :]
        tmin = jnp.min(d, axis=1)
        ids = jax.lax.broadcasted_iota(jnp.int32, (_T_BLK, _K_BLK), 1) + koff
        tidx = jnp.min(jnp.where(d == tmin[:, None], ids, jnp.int32(2**30)),
                       axis=1)
        upd = tmin < best_d
        return jnp.where(upd, tmin, best_d), jnp.where(upd, tidx, best_i)

    best_d = jnp.full((_T_BLK,), jnp.inf, jnp.float32)
    best_i = jnp.zeros((_T_BLK,), jnp.int32)
    best_d, best_i = jax.lax.fori_loop(0, n_tiles, dist_step,
                                       (best_d, best_i))

    idx_ref[...] = best_i.reshape(1, 1, _T_BLK)

    def gather_step(kt, zq):
        koff = kt * _K_BLK
        c = cb_ref[pl.ds(koff, _K_BLK), :]
        ids = jax.lax.broadcasted_iota(jnp.int32, (_T_BLK, _K_BLK), 1) + koff
        oh = (best_i[:, None] == ids).astype(jnp.float32)
        return zq + jax.lax.dot_general(
            oh, c, (((1,), (0,)), ((), ())),
            preferred_element_type=jnp.float32)

    zq = jax.lax.fori_loop(0, n_tiles, gather_step,
                           jnp.zeros((_T_BLK, _CODE_DIM), jnp.float32))
    zq_ref[...] = z + (zq - z)

    @pl.when(pl.program_id(0) == 0)
    def _():
        dsum_ref[0, 0] = 0.0
    dsum_ref[0, 0] += jnp.sum(best_d)


def kernel(z_e, codebook):
    b, t, d = z_e.shape
    n_tok = b * t
    grid = n_tok // _T_BLK
    z = z_e.reshape(n_tok, d)

    zq_st, idx3, dsum = pl.pallas_call(
        _vq_body,
        grid=(grid,),
        in_specs=[
            pl.BlockSpec((_T_BLK, d), lambda i: (i, 0)),
            pl.BlockSpec((_N_CODES, d), lambda i: (0, 0)),
        ],
        out_specs=[
            pl.BlockSpec((_T_BLK, d), lambda i: (i, 0)),
            pl.BlockSpec((1, 1, _T_BLK), lambda i: (i, 0, 0)),
            pl.BlockSpec(memory_space=pltpu.SMEM),
        ],
        out_shape=[
            jax.ShapeDtypeStruct((n_tok, d), jnp.float32),
            jax.ShapeDtypeStruct((grid, 1, _T_BLK), jnp.int32),
            jax.ShapeDtypeStruct((1, 1), jnp.float32),
        ],
    )(z, codebook)

    mean_d = dsum[0, 0] / (n_tok * d)
    loss = mean_d + _BETA * mean_d
    return (zq_st.reshape(b, t, d), idx3.reshape(b, t), loss)
